# baseline (speedup 1.0000x reference)
import functools

import jax
import jax.numpy as jnp
from jax import lax
from jax.experimental import pallas as pl
from jax.experimental.pallas import tpu as pltpu
from jax.experimental.pallas import tpu_sc as plsc

N = 10000
NPAD = 10240
E = 320000
IN = 128
HID = 16
H = 8
OUT = 64
D1 = H * HID

K = 32
NC = 2
NS = 16
EDGES = E + N
EPAD = 344064
IDXROWS = EPAD // K + 8
ROWS_PER_TILE = NPAD // NS
SB = K * 128 * 4

_GDN = lax.GatherDimensionNumbers(
    offset_dims=(), collapsed_slice_dims=(0,), start_index_map=(0,))


def _xg(v, idx):
    return lax.gather(v, idx[:, None], _GDN, (1,),
                      mode=lax.GatherScatterMode.PROMISE_IN_BOUNDS)


def _xor_idx(sh):
    return jnp.bitwise_xor(lax.iota(jnp.int32, 16), sh)


def _lanesum(v):
    for sh in (8, 4, 2, 1):
        v = v + _xg(v, _xor_idx(sh))
    return v


def _drain(dummy_src, dst_ref, sem):
    pltpu.make_async_copy(dummy_src, dst_ref, sem).wait()


def _make_edge_kernel(head_split, G):
    nchunks = EPAD // (NS * K) if head_split else EPAD // (NC * NS * K)
    assert nchunks % (2 * G) == 0
    pairs = nchunks // (2 * G)
    mesh = plsc.VectorSubcoreMesh(core_axis_name="c", subcore_axis_name="s")

    @functools.partial(
        pl.kernel,
        out_type=jax.ShapeDtypeStruct((NC, NPAD, 128), jnp.float32),
        mesh=mesh,
        scratch_types=[
            pltpu.VMEM((2, K, 128), jnp.float32),
            pltpu.VMEM((2, K, 128), jnp.float32),
            pltpu.VMEM((2, K, 128), jnp.float32),
            pltpu.VMEM((2, G, K), jnp.int32),
            pltpu.VMEM((2, G, K), jnp.int32),
            pltpu.VMEM((128,), jnp.float32),
            pltpu.VMEM((K,), jnp.int32),
            pltpu.VMEM_SHARED((NPAD, 128), jnp.float32),
            [pltpu.SemaphoreType.DMA] * 2,
            [pltpu.SemaphoreType.DMA] * 2,
            [pltpu.SemaphoreType.DMA] * 2,
            [pltpu.SemaphoreType.DMA] * 2,
            [pltpu.SemaphoreType.DMA] * 2,
        ],
    )
    def k(xl_hbm, xr_hbm, src_hbm, dst_hbm, att_hbm, out_hbm,
          xlv, xrv, msg, srcb, dstb, attv, padidx, acc,
          sxl, sxr, ssc, sis, sid_sem):
        cid = lax.axis_index("c")
        sid = lax.axis_index("s")

        pltpu.sync_copy(att_hbm, attv)

        zero16 = jnp.zeros((16,), jnp.float32)

        def zrow(r, carry):
            for b in range(2):
                for j in range(8):
                    msg[b, r, pl.ds(j * 16, 16)] = zero16
            return carry

        lax.fori_loop(0, K, zrow, 0)

        row0 = sid * ROWS_PER_TILE
        for j in range(ROWS_PER_TILE // K):
            pltpu.sync_copy(msg.at[0], acc.at[pl.ds(row0 + j * K, K)])
        plsc.subcore_barrier()

        padn = jnp.full((16,), N, jnp.int32)
        for j in range(K // 16):
            padidx[pl.ds(j * 16, 16)] = padn
        pltpu.async_copy(msg.at[0], acc.at[padidx], ssc[0], add=True)
        pltpu.async_copy(msg.at[1], acc.at[padidx], ssc[1], add=True)

        lanes16 = lax.iota(jnp.int32, 16)
        if head_split:
            coff = cid * 64
            att_vecs = [attv[pl.ds(coff + hl * 16, 16)] for hl in range(4)]
            x8, x4, x2, x1 = (_xor_idx(sh) for sh in (8, 4, 2, 1))
            bidx = [jnp.full((16,), 4 * hl, jnp.int32) for hl in range(4)]
            didx = jnp.where(lanes16 < 4, lanes16 * 4, 0)
            mask4 = jnp.where(lanes16 < 4, jnp.float32(1.0), jnp.float32(0.0))
            m4 = lanes16 < 4
            m8 = lanes16 < 8
            m12 = lanes16 < 12
            rowbase = sid * nchunks

            def edge_body(b, e):
                rs = []
                avs = []
                for hl in range(4):
                    a = xlv[b, e, pl.ds(coff + hl * 16, 16)]
                    bb = xrv[b, e, pl.ds(coff + hl * 16, 16)]
                    t = a + bb
                    t = jnp.maximum(t, t * 0.2)
                    s = t * att_vecs[hl]
                    s = s + _xg(s, x8)
                    s = s + _xg(s, x4)
                    rs.append(s)
                    avs.append(a)
                w = jnp.where(m8, jnp.where(m4, rs[0], rs[1]),
                              jnp.where(m12, rs[2], rs[3]))
                w = w + _xg(w, x2)
                w = w + _xg(w, x1)
                pw = jnp.exp(w)
                for hl in range(4):
                    pv = _xg(pw, bidx[hl])
                    msg[b, e, pl.ds(hl * 16, 16)] = avs[hl] * pv
                msg[b, e, pl.ds(64, 16)] = _xg(pw, didx) * mask4
        else:
            att_vecs = [attv[pl.ds(j * 16, 16)] for j in range(4)]
            unit0 = jnp.where(lax.iota(jnp.int32, 16) == 0,
                              jnp.float32(1.0), jnp.float32(0.0))
            rowbase = (cid * NS + sid) * nchunks

            def edge_body(b, e):
                avals = []
                s = None
                for j in range(4):
                    a = xlv[b, e, pl.ds(j * 16, 16)]
                    bb = xrv[b, e, pl.ds(j * 16, 16)]
                    avals.append(a)
                    t = a + bb
                    t = jnp.maximum(t, t * 0.2)
                    sj = t * att_vecs[j]
                    s = sj if s is None else s + sj
                pv = jnp.exp(_lanesum(s))
                for j in range(4):
                    msg[b, e, pl.ds(j * 16, 16)] = avals[j] * pv
                msg[b, e, pl.ds(64, 16)] = pv * unit0

        def issue_idx(sset, srow):
            pltpu.async_copy(src_hbm.at[pl.ds(srow, G)], srcb.at[sset],
                             sis[sset])
            pltpu.async_copy(dst_hbm.at[pl.ds(srow, G)], dstb.at[sset],
                             sid_sem[sset])

        def issue_gather(nb, nset, nrow):
            pltpu.async_copy(xl_hbm.at[srcb.at[nset, nrow]], xlv.at[nb],
                             sxl[nb])
            pltpu.async_copy(xr_hbm.at[dstb.at[nset, nrow]], xrv.at[nb],
                             sxr[nb])

        pltpu.sync_copy(src_hbm.at[pl.ds(rowbase, G)], srcb.at[0])
        pltpu.sync_copy(dst_hbm.at[pl.ds(rowbase, G)], dstb.at[0])
        issue_gather(0, 0, 0)

        hdummy = xl_hbm.at[pl.ds(0, K)]
        idummy = src_hbm.at[pl.ds(0, G)]

        def pair(s2, carry):
            for sp in range(2):
                s = 2 * s2 + sp
                for j in range(G):
                    b = j % 2
                    if j == 0:
                        issue_idx(1 - sp, rowbase + (s + 1) * G)
                    if j == 2:
                        _drain(idummy, srcb.at[1 - sp], sis[1 - sp])
                        _drain(idummy, dstb.at[1 - sp], sid_sem[1 - sp])
                    nb = 1 - b
                    nset, nrow = (sp, j + 1) if j < G - 1 else (1 - sp, 0)
                    issue_gather(nb, nset, nrow)
                    _drain(hdummy, xlv.at[b], sxl[b])
                    _drain(hdummy, xrv.at[b], sxr[b])
                    _drain(hdummy, msg.at[b], ssc[b])
                    @plsc.parallel_loop(0, K, unroll=8)
                    def _edges(e, _b=b):
                        edge_body(_b, e)
                    pltpu.async_copy(msg.at[b], acc.at[dstb.at[sp, j]],
                                     ssc[b], add=True)
            return carry

        lax.fori_loop(0, pairs, pair, 0)

        _drain(hdummy, xlv.at[0], sxl[0])
        _drain(hdummy, xrv.at[0], sxr[0])
        _drain(hdummy, msg.at[0], ssc[0])
        _drain(hdummy, msg.at[1], ssc[1])

        plsc.subcore_barrier()
        for j in range(ROWS_PER_TILE // K):
            sl = pl.ds(row0 + j * K, K)
            pltpu.sync_copy(acc.at[sl], out_hbm.at[cid].at[sl])

    return k


def _transform(x, Wl, Wr):
    Din, D = Wl.shape
    RB = 256

    def body(x_ref, wl_ref, wr_ref, xl_ref, xr_ref):
        xb = x_ref[...]
        xl_ref[...] = jnp.dot(xb, wl_ref[...], preferred_element_type=jnp.float32)
        xr_ref[...] = jnp.dot(xb, wr_ref[...], preferred_element_type=jnp.float32)

    return pl.pallas_call(
        body,
        grid=(NPAD // RB,),
        in_specs=[
            pl.BlockSpec((RB, Din), lambda i: (i, 0)),
            pl.BlockSpec((Din, D), lambda i: (0, 0)),
            pl.BlockSpec((Din, D), lambda i: (0, 0)),
        ],
        out_specs=[
            pl.BlockSpec((RB, D), lambda i: (i, 0)),
            pl.BlockSpec((RB, D), lambda i: (i, 0)),
        ],
        out_shape=[
            jax.ShapeDtypeStruct((NPAD, D), jnp.float32),
            jax.ShapeDtypeStruct((NPAD, D), jnp.float32),
        ],
    )(x, Wl, Wr)


def _combine_transform(acc0, acc1, expand, b, Wl, Wr):
    Dn = Wl.shape[1]
    RB = 256

    def body(a0_ref, a1_ref, ex_ref, b_ref, wl_ref, wr_ref, xl_ref, xr_ref):
        a0 = a0_ref[...]
        a1 = a1_ref[...]
        num = jnp.concatenate([a0[:, :64], a1[:, :64]], axis=1)
        den = jnp.concatenate([a0[:, 64:68], a1[:, 64:68]], axis=1)
        dexp = jnp.dot(den, ex_ref[...], preferred_element_type=jnp.float32)
        hh = num / (dexp + 1e-16) + b_ref[...]
        hh = jnp.where(hh > 0, hh, jnp.exp(jnp.minimum(hh, 0.0)) - 1.0)
        xl_ref[...] = jnp.dot(hh, wl_ref[...], preferred_element_type=jnp.float32)
        xr_ref[...] = jnp.dot(hh, wr_ref[...], preferred_element_type=jnp.float32)

    return pl.pallas_call(
        body,
        grid=(NPAD // RB,),
        in_specs=[
            pl.BlockSpec((RB, 128), lambda i: (i, 0)),
            pl.BlockSpec((RB, 128), lambda i: (i, 0)),
            pl.BlockSpec((H, D1), lambda i: (0, 0)),
            pl.BlockSpec((1, D1), lambda i: (0, 0)),
            pl.BlockSpec((D1, Dn), lambda i: (0, 0)),
            pl.BlockSpec((D1, Dn), lambda i: (0, 0)),
        ],
        out_specs=[
            pl.BlockSpec((RB, Dn), lambda i: (i, 0)),
            pl.BlockSpec((RB, Dn), lambda i: (i, 0)),
        ],
        out_shape=[
            jax.ShapeDtypeStruct((NPAD, Dn), jnp.float32),
            jax.ShapeDtypeStruct((NPAD, Dn), jnp.float32),
        ],
    )(acc0, acc1, expand, b.reshape(1, D1), Wl, Wr)


def _combine_final(acc0, acc1, b):
    RB = 256

    def body(a0_ref, a1_ref, b_ref, o_ref):
        a0 = a0_ref[...]
        a1 = a1_ref[...]
        num = a0[:, :OUT] + a1[:, :OUT]
        den = a0[:, OUT:OUT + 1] + a1[:, OUT:OUT + 1]
        dexp = jnp.broadcast_to(den, (RB, OUT))
        o_ref[...] = num / (dexp + 1e-16) + b_ref[...]

    return pl.pallas_call(
        body,
        grid=(NPAD // RB,),
        in_specs=[
            pl.BlockSpec((RB, 128), lambda i: (i, 0)),
            pl.BlockSpec((RB, 128), lambda i: (i, 0)),
            pl.BlockSpec((1, OUT), lambda i: (0, 0)),
        ],
        out_specs=pl.BlockSpec((RB, OUT), lambda i: (i, 0)),
        out_shape=jax.ShapeDtypeStruct((NPAD, OUT), jnp.float32),
    )(acc0, acc1, b.reshape(1, OUT))


def kernel(x, edge_index, Wl1, Wr1, att1, b1, Wl2, Wr2, att2, b2,
           Wl3, Wr3, att3, b3):
    loop = jnp.arange(N, dtype=edge_index.dtype)
    padv = N + jnp.arange(IDXROWS * K - EDGES, dtype=edge_index.dtype) % (NPAD - N)
    src = jnp.concatenate([edge_index[0], loop, padv]).reshape(IDXROWS, K)
    dst = jnp.concatenate([edge_index[1], loop, padv]).reshape(IDXROWS, K)
    x_pad = jnp.concatenate([x, jnp.zeros((NPAD - N, IN), x.dtype)], axis=0)

    expand8 = jnp.kron(jnp.eye(H, dtype=jnp.float32),
                       jnp.ones((1, HID), dtype=jnp.float32))

    edge_h = _make_edge_kernel(True, 8)
    edge_s = _make_edge_kernel(False, 8)

    xl, xr = _transform(x_pad, Wl1, Wr1)
    acc = edge_h(xl, xr, src, dst, att1.reshape(-1))
    xl, xr = _combine_transform(acc[0], acc[1], expand8, b1, Wl2, Wr2)
    acc = edge_h(xl, xr, src, dst, att2.reshape(-1))
    wpad = jnp.zeros((D1, D1 - OUT), jnp.float32)
    xl, xr = _combine_transform(acc[0], acc[1], expand8, b2,
                                jnp.concatenate([Wl3, wpad], axis=1),
                                jnp.concatenate([Wr3, wpad], axis=1))
    att3p = jnp.concatenate([att3.reshape(-1), jnp.zeros((64,), jnp.float32)])
    acc = edge_s(xl, xr, src, dst, att3p)
    out = _combine_final(acc[0], acc[1], b3)
    return out[:N]


# speedup vs baseline: 1.4232x; 1.4232x over previous
import functools

import jax
import jax.numpy as jnp
from jax import lax
from jax.experimental import pallas as pl
from jax.experimental.pallas import tpu as pltpu
from jax.experimental.pallas import tpu_sc as plsc

N = 10000
NPAD = 10240
E = 320000
IN = 128
HID = 16
H = 8
OUT = 64
D1 = H * HID

K = 32
NC = 2
NS = 16
EDGES = E + N
EPAD = 344064
IDXROWS = EPAD // K + 8
ROWS_PER_TILE = NPAD // NS
SB = K * 128 * 4

_GDN = lax.GatherDimensionNumbers(
    offset_dims=(), collapsed_slice_dims=(0,), start_index_map=(0,))


def _xg(v, idx):
    return lax.gather(v, idx[:, None], _GDN, (1,),
                      mode=lax.GatherScatterMode.PROMISE_IN_BOUNDS)


def _xor_idx(sh):
    return jnp.bitwise_xor(lax.iota(jnp.int32, 16), sh)


def _lanesum(v):
    for sh in (8, 4, 2, 1):
        v = v + _xg(v, _xor_idx(sh))
    return v


def _drain(dummy_src, dst_ref, sem):
    pltpu.make_async_copy(dummy_src, dst_ref, sem).wait()


def _make_edge_kernel(head_split, G):
    nchunks = EPAD // (NS * K) if head_split else EPAD // (NC * NS * K)
    assert nchunks % (2 * G) == 0
    pairs = nchunks // (2 * G)
    mesh = plsc.VectorSubcoreMesh(core_axis_name="c", subcore_axis_name="s")

    @functools.partial(
        pl.kernel,
        out_type=jax.ShapeDtypeStruct((NC, NPAD, 128), jnp.float32),
        mesh=mesh,
        scratch_types=[
            pltpu.VMEM((2, K, 128), jnp.float32),
            pltpu.VMEM((2, K, 128), jnp.float32),
            pltpu.VMEM((2, K, 128), jnp.float32),
            pltpu.VMEM((2, G, K), jnp.int32),
            pltpu.VMEM((2, G, K), jnp.int32),
            pltpu.VMEM((128,), jnp.float32),
            pltpu.VMEM((K,), jnp.int32),
            pltpu.VMEM_SHARED((NPAD, 128), jnp.float32),
            [pltpu.SemaphoreType.DMA] * 2,
            [pltpu.SemaphoreType.DMA] * 2,
            [pltpu.SemaphoreType.DMA] * 2,
            [pltpu.SemaphoreType.DMA] * 2,
            [pltpu.SemaphoreType.DMA] * 2,
        ],
    )
    def k(xl_hbm, xr_hbm, src_hbm, dst_hbm, att_hbm, out_hbm,
          xlv, xrv, msg, srcb, dstb, attv, padidx, acc,
          sxl, sxr, ssc, sis, sid_sem):
        cid = lax.axis_index("c")
        sid = lax.axis_index("s")

        pltpu.sync_copy(att_hbm, attv)

        zero16 = jnp.zeros((16,), jnp.float32)

        def zrow(r, carry):
            for b in range(2):
                for j in range(8):
                    msg[b, r, pl.ds(j * 16, 16)] = zero16
            return carry

        lax.fori_loop(0, K, zrow, 0)

        row0 = sid * ROWS_PER_TILE
        for j in range(ROWS_PER_TILE // K):
            pltpu.sync_copy(msg.at[0], acc.at[pl.ds(row0 + j * K, K)])
        plsc.subcore_barrier()

        padn = jnp.full((16,), N, jnp.int32)
        for j in range(K // 16):
            padidx[pl.ds(j * 16, 16)] = padn
        pltpu.async_copy(msg.at[0], acc.at[padidx], ssc[0], add=True)
        pltpu.async_copy(msg.at[1], acc.at[padidx], ssc[1], add=True)

        lanes16 = lax.iota(jnp.int32, 16)
        if head_split:
            coff = cid * 64
            att_vecs = [attv[pl.ds(coff + hl * 16, 16)] for hl in range(4)]
            x8, x4, x2, x1 = (_xor_idx(sh) for sh in (8, 4, 2, 1))
            bidx = [jnp.full((16,), 4 * hl, jnp.int32) for hl in range(4)]
            didx = jnp.where(lanes16 < 4, lanes16 * 4, 0)
            mask4 = jnp.where(lanes16 < 4, jnp.float32(1.0), jnp.float32(0.0))
            m4 = lanes16 < 4
            m8 = lanes16 < 8
            m12 = lanes16 < 12
            rowbase = sid * nchunks

            def edge_body(b, e):
                rs = []
                avs = []
                for hl in range(4):
                    a = xlv[b, e, pl.ds(coff + hl * 16, 16)]
                    bb = xrv[b, e, pl.ds(coff + hl * 16, 16)]
                    t = a + bb
                    t = jnp.maximum(t, t * 0.2)
                    s = t * att_vecs[hl]
                    s = s + _xg(s, x8)
                    s = s + _xg(s, x4)
                    rs.append(s)
                    avs.append(a)
                w = jnp.where(m8, jnp.where(m4, rs[0], rs[1]),
                              jnp.where(m12, rs[2], rs[3]))
                w = w + _xg(w, x2)
                w = w + _xg(w, x1)
                pw = jnp.exp(w)
                for hl in range(4):
                    pv = _xg(pw, bidx[hl])
                    msg[b, e, pl.ds(hl * 16, 16)] = avs[hl] * pv
                msg[b, e, pl.ds(64, 16)] = _xg(pw, didx) * mask4
        else:
            att_vecs = [attv[pl.ds(j * 16, 16)] for j in range(4)]
            unit0 = jnp.where(lax.iota(jnp.int32, 16) == 0,
                              jnp.float32(1.0), jnp.float32(0.0))
            rowbase = (cid * NS + sid) * nchunks

            def edge_body(b, e):
                avals = []
                s = None
                for j in range(4):
                    a = xlv[b, e, pl.ds(j * 16, 16)]
                    bb = xrv[b, e, pl.ds(j * 16, 16)]
                    avals.append(a)
                    t = a + bb
                    t = jnp.maximum(t, t * 0.2)
                    sj = t * att_vecs[j]
                    s = sj if s is None else s + sj
                pv = jnp.exp(_lanesum(s))
                for j in range(4):
                    msg[b, e, pl.ds(j * 16, 16)] = avals[j] * pv
                msg[b, e, pl.ds(64, 16)] = pv * unit0

        def issue_idx(sset, srow):
            pltpu.async_copy(src_hbm.at[pl.ds(srow, G)], srcb.at[sset],
                             sis[sset])
            pltpu.async_copy(dst_hbm.at[pl.ds(srow, G)], dstb.at[sset],
                             sid_sem[sset])

        def issue_gather(nb, nset, nrow):
            pltpu.async_copy(xl_hbm.at[srcb.at[nset, nrow]], xlv.at[nb],
                             sxl[nb])
            pltpu.async_copy(xr_hbm.at[dstb.at[nset, nrow]], xrv.at[nb],
                             sxr[nb])

        pltpu.sync_copy(src_hbm.at[pl.ds(rowbase, G)], srcb.at[0])
        pltpu.sync_copy(dst_hbm.at[pl.ds(rowbase, G)], dstb.at[0])
        issue_gather(0, 0, 0)

        hdummy = xl_hbm.at[pl.ds(0, K)]
        idummy = src_hbm.at[pl.ds(0, G)]

        def pair(s2, carry):
            for sp in range(2):
                s = 2 * s2 + sp
                for j in range(G):
                    b = j % 2
                    if j == 0:
                        issue_idx(1 - sp, rowbase + (s + 1) * G)
                    if j == 2:
                        _drain(idummy, srcb.at[1 - sp], sis[1 - sp])
                        _drain(idummy, dstb.at[1 - sp], sid_sem[1 - sp])
                    nb = 1 - b
                    nset, nrow = (sp, j + 1) if j < G - 1 else (1 - sp, 0)
                    issue_gather(nb, nset, nrow)
                    _drain(hdummy, xlv.at[b], sxl[b])
                    _drain(hdummy, xrv.at[b], sxr[b])
                    _drain(hdummy, msg.at[b], ssc[b])
                    @plsc.parallel_loop(0, K, unroll=2)
                    def _edges(e, _b=b):
                        edge_body(_b, e)
                    pltpu.async_copy(msg.at[b], acc.at[dstb.at[sp, j]],
                                     ssc[b], add=True)
            return carry

        lax.fori_loop(0, pairs, pair, 0)

        _drain(hdummy, xlv.at[0], sxl[0])
        _drain(hdummy, xrv.at[0], sxr[0])
        _drain(hdummy, msg.at[0], ssc[0])
        _drain(hdummy, msg.at[1], ssc[1])

        plsc.subcore_barrier()
        for j in range(ROWS_PER_TILE // K):
            sl = pl.ds(row0 + j * K, K)
            pltpu.sync_copy(acc.at[sl], out_hbm.at[cid].at[sl])

    return k


def _transform(x, Wl, Wr):
    Din, D = Wl.shape
    RB = 256

    def body(x_ref, wl_ref, wr_ref, xl_ref, xr_ref):
        xb = x_ref[...]
        xl_ref[...] = jnp.dot(xb, wl_ref[...], preferred_element_type=jnp.float32)
        xr_ref[...] = jnp.dot(xb, wr_ref[...], preferred_element_type=jnp.float32)

    return pl.pallas_call(
        body,
        grid=(NPAD // RB,),
        in_specs=[
            pl.BlockSpec((RB, Din), lambda i: (i, 0)),
            pl.BlockSpec((Din, D), lambda i: (0, 0)),
            pl.BlockSpec((Din, D), lambda i: (0, 0)),
        ],
        out_specs=[
            pl.BlockSpec((RB, D), lambda i: (i, 0)),
            pl.BlockSpec((RB, D), lambda i: (i, 0)),
        ],
        out_shape=[
            jax.ShapeDtypeStruct((NPAD, D), jnp.float32),
            jax.ShapeDtypeStruct((NPAD, D), jnp.float32),
        ],
    )(x, Wl, Wr)


def _combine_transform(acc0, acc1, expand, b, Wl, Wr):
    Dn = Wl.shape[1]
    RB = 256

    def body(a0_ref, a1_ref, ex_ref, b_ref, wl_ref, wr_ref, xl_ref, xr_ref):
        a0 = a0_ref[...]
        a1 = a1_ref[...]
        num = jnp.concatenate([a0[:, :64], a1[:, :64]], axis=1)
        den = jnp.concatenate([a0[:, 64:68], a1[:, 64:68]], axis=1)
        dexp = jnp.dot(den, ex_ref[...], preferred_element_type=jnp.float32)
        hh = num / (dexp + 1e-16) + b_ref[...]
        hh = jnp.where(hh > 0, hh, jnp.exp(jnp.minimum(hh, 0.0)) - 1.0)
        xl_ref[...] = jnp.dot(hh, wl_ref[...], preferred_element_type=jnp.float32)
        xr_ref[...] = jnp.dot(hh, wr_ref[...], preferred_element_type=jnp.float32)

    return pl.pallas_call(
        body,
        grid=(NPAD // RB,),
        in_specs=[
            pl.BlockSpec((RB, 128), lambda i: (i, 0)),
            pl.BlockSpec((RB, 128), lambda i: (i, 0)),
            pl.BlockSpec((H, D1), lambda i: (0, 0)),
            pl.BlockSpec((1, D1), lambda i: (0, 0)),
            pl.BlockSpec((D1, Dn), lambda i: (0, 0)),
            pl.BlockSpec((D1, Dn), lambda i: (0, 0)),
        ],
        out_specs=[
            pl.BlockSpec((RB, Dn), lambda i: (i, 0)),
            pl.BlockSpec((RB, Dn), lambda i: (i, 0)),
        ],
        out_shape=[
            jax.ShapeDtypeStruct((NPAD, Dn), jnp.float32),
            jax.ShapeDtypeStruct((NPAD, Dn), jnp.float32),
        ],
    )(acc0, acc1, expand, b.reshape(1, D1), Wl, Wr)


def _combine_final(acc0, acc1, b):
    RB = 256

    def body(a0_ref, a1_ref, b_ref, o_ref):
        a0 = a0_ref[...]
        a1 = a1_ref[...]
        num = a0[:, :OUT] + a1[:, :OUT]
        den = a0[:, OUT:OUT + 1] + a1[:, OUT:OUT + 1]
        dexp = jnp.broadcast_to(den, (RB, OUT))
        o_ref[...] = num / (dexp + 1e-16) + b_ref[...]

    return pl.pallas_call(
        body,
        grid=(NPAD // RB,),
        in_specs=[
            pl.BlockSpec((RB, 128), lambda i: (i, 0)),
            pl.BlockSpec((RB, 128), lambda i: (i, 0)),
            pl.BlockSpec((1, OUT), lambda i: (0, 0)),
        ],
        out_specs=pl.BlockSpec((RB, OUT), lambda i: (i, 0)),
        out_shape=jax.ShapeDtypeStruct((NPAD, OUT), jnp.float32),
    )(acc0, acc1, b.reshape(1, OUT))


def kernel(x, edge_index, Wl1, Wr1, att1, b1, Wl2, Wr2, att2, b2,
           Wl3, Wr3, att3, b3):
    loop = jnp.arange(N, dtype=edge_index.dtype)
    padv = N + jnp.arange(IDXROWS * K - EDGES, dtype=edge_index.dtype) % (NPAD - N)
    src = jnp.concatenate([edge_index[0], loop, padv]).reshape(IDXROWS, K)
    dst = jnp.concatenate([edge_index[1], loop, padv]).reshape(IDXROWS, K)
    x_pad = jnp.concatenate([x, jnp.zeros((NPAD - N, IN), x.dtype)], axis=0)

    expand8 = jnp.kron(jnp.eye(H, dtype=jnp.float32),
                       jnp.ones((1, HID), dtype=jnp.float32))

    edge_h = _make_edge_kernel(True, 8)
    edge_s = _make_edge_kernel(False, 8)

    xl, xr = _transform(x_pad, Wl1, Wr1)
    acc = edge_h(xl, xr, src, dst, att1.reshape(-1))
    xl, xr = _combine_transform(acc[0], acc[1], expand8, b1, Wl2, Wr2)
    acc = edge_h(xl, xr, src, dst, att2.reshape(-1))
    wpad = jnp.zeros((D1, D1 - OUT), jnp.float32)
    xl, xr = _combine_transform(acc[0], acc[1], expand8, b2,
                                jnp.concatenate([Wl3, wpad], axis=1),
                                jnp.concatenate([Wr3, wpad], axis=1))
    att3p = jnp.concatenate([att3.reshape(-1), jnp.zeros((64,), jnp.float32)])
    acc = edge_s(xl, xr, src, dst, att3p)
    out = _combine_final(acc[0], acc[1], b3)
    return out[:N]


# speedup vs baseline: 1.4306x; 1.0052x over previous
import functools

import jax
import jax.numpy as jnp
from jax import lax
from jax.experimental import pallas as pl
from jax.experimental.pallas import tpu as pltpu
from jax.experimental.pallas import tpu_sc as plsc

N = 10000
NPAD = 10240
E = 320000
IN = 128
HID = 16
H = 8
OUT = 64
D1 = H * HID

K = 32
NC = 2
NS = 16
EDGES = E + N
EPAD = 344064
IDXROWS = EPAD // K + 8
ROWS_PER_TILE = NPAD // NS
SB = K * 128 * 4

_GDN = lax.GatherDimensionNumbers(
    offset_dims=(), collapsed_slice_dims=(0,), start_index_map=(0,))


def _xg(v, idx):
    return lax.gather(v, idx[:, None], _GDN, (1,),
                      mode=lax.GatherScatterMode.PROMISE_IN_BOUNDS)


def _xor_idx(sh):
    return jnp.bitwise_xor(lax.iota(jnp.int32, 16), sh)


def _lanesum(v):
    for sh in (8, 4, 2, 1):
        v = v + _xg(v, _xor_idx(sh))
    return v


def _drain(dummy_src, dst_ref, sem):
    pltpu.make_async_copy(dummy_src, dst_ref, sem).wait()


def _make_edge_kernel(head_split, G):
    nchunks = EPAD // (NS * K) if head_split else EPAD // (NC * NS * K)
    assert nchunks % (2 * G) == 0
    pairs = nchunks // (2 * G)
    mesh = plsc.VectorSubcoreMesh(core_axis_name="c", subcore_axis_name="s")

    @functools.partial(
        pl.kernel,
        out_type=jax.ShapeDtypeStruct((NC, NPAD, 128), jnp.float32),
        mesh=mesh,
        scratch_types=[
            pltpu.VMEM((2, K, 128), jnp.float32),
            pltpu.VMEM((2, K, 128), jnp.float32),
            pltpu.VMEM((2, K, 128), jnp.float32),
            pltpu.VMEM((2, G, K), jnp.int32),
            pltpu.VMEM((2, G, K), jnp.int32),
            pltpu.VMEM((128,), jnp.float32),
            pltpu.VMEM((K,), jnp.int32),
            pltpu.VMEM_SHARED((NPAD, 128), jnp.float32),
            [pltpu.SemaphoreType.DMA] * 2,
            [pltpu.SemaphoreType.DMA] * 2,
            [pltpu.SemaphoreType.DMA] * 2,
            [pltpu.SemaphoreType.DMA] * 2,
            [pltpu.SemaphoreType.DMA] * 2,
        ],
    )
    def k(xl_hbm, xr_hbm, src_hbm, dst_hbm, att_hbm, out_hbm,
          xlv, xrv, msg, srcb, dstb, attv, padidx, acc,
          sxl, sxr, ssc, sis, sid_sem):
        cid = lax.axis_index("c")
        sid = lax.axis_index("s")

        pltpu.sync_copy(att_hbm, attv)

        zero16 = jnp.zeros((16,), jnp.float32)

        def zrow(r, carry):
            for b in range(2):
                for j in range(8):
                    msg[b, r, pl.ds(j * 16, 16)] = zero16
            return carry

        lax.fori_loop(0, K, zrow, 0)

        row0 = sid * ROWS_PER_TILE
        for j in range(ROWS_PER_TILE // K):
            pltpu.sync_copy(msg.at[0], acc.at[pl.ds(row0 + j * K, K)])
        plsc.subcore_barrier()

        padn = jnp.full((16,), N, jnp.int32)
        for j in range(K // 16):
            padidx[pl.ds(j * 16, 16)] = padn
        pltpu.async_copy(msg.at[0], acc.at[padidx], ssc[0], add=True)
        pltpu.async_copy(msg.at[1], acc.at[padidx], ssc[1], add=True)

        lanes16 = lax.iota(jnp.int32, 16)
        if head_split:
            coff = cid * 64
            att_vecs = [attv[pl.ds(coff + hl * 16, 16)] for hl in range(4)]
            x8, x4, x2, x1 = (_xor_idx(sh) for sh in (8, 4, 2, 1))
            bidx = [jnp.full((16,), 4 * hl, jnp.int32) for hl in range(4)]
            didx = jnp.where(lanes16 < 4, lanes16 * 4, 0)
            mask4 = jnp.where(lanes16 < 4, jnp.float32(1.0), jnp.float32(0.0))
            m4 = lanes16 < 4
            m8 = lanes16 < 8
            m12 = lanes16 < 12
            rowbase = sid * nchunks

            def edge_body(b, e):
                rs = []
                avs = []
                for hl in range(4):
                    a = xlv[b, e, pl.ds(coff + hl * 16, 16)]
                    bb = xrv[b, e, pl.ds(coff + hl * 16, 16)]
                    t = a + bb
                    t = jnp.maximum(t, t * 0.2)
                    s = t * att_vecs[hl]
                    s = s + _xg(s, x8)
                    s = s + _xg(s, x4)
                    rs.append(s)
                    avs.append(a)
                w = jnp.where(m8, jnp.where(m4, rs[0], rs[1]),
                              jnp.where(m12, rs[2], rs[3]))
                w = w + _xg(w, x2)
                w = w + _xg(w, x1)
                pw = jnp.exp(w)
                for hl in range(4):
                    pv = _xg(pw, bidx[hl])
                    msg[b, e, pl.ds(hl * 16, 16)] = avs[hl] * pv
                msg[b, e, pl.ds(64, 16)] = _xg(pw, didx) * mask4
        else:
            att_vecs = [attv[pl.ds(j * 16, 16)] for j in range(4)]
            unit0 = jnp.where(lax.iota(jnp.int32, 16) == 0,
                              jnp.float32(1.0), jnp.float32(0.0))
            rowbase = (cid * NS + sid) * nchunks

            def edge_body(b, e):
                avals = []
                s = None
                for j in range(4):
                    a = xlv[b, e, pl.ds(j * 16, 16)]
                    bb = xrv[b, e, pl.ds(j * 16, 16)]
                    avals.append(a)
                    t = a + bb
                    t = jnp.maximum(t, t * 0.2)
                    sj = t * att_vecs[j]
                    s = sj if s is None else s + sj
                pv = jnp.exp(_lanesum(s))
                for j in range(4):
                    msg[b, e, pl.ds(j * 16, 16)] = avals[j] * pv
                msg[b, e, pl.ds(64, 16)] = pv * unit0

        def issue_idx(sset, srow):
            pltpu.async_copy(src_hbm.at[pl.ds(srow, G)], srcb.at[sset],
                             sis[sset])
            pltpu.async_copy(dst_hbm.at[pl.ds(srow, G)], dstb.at[sset],
                             sid_sem[sset])

        def issue_gather(nb, nset, nrow):
            pltpu.async_copy(xl_hbm.at[srcb.at[nset, nrow]], xlv.at[nb],
                             sxl[nb])
            pltpu.async_copy(xr_hbm.at[dstb.at[nset, nrow]], xrv.at[nb],
                             sxr[nb])

        pltpu.sync_copy(src_hbm.at[pl.ds(rowbase, G)], srcb.at[0])
        pltpu.sync_copy(dst_hbm.at[pl.ds(rowbase, G)], dstb.at[0])
        issue_gather(0, 0, 0)

        hdummy = xl_hbm.at[pl.ds(0, K)]
        idummy = src_hbm.at[pl.ds(0, G)]

        def pair(s2, carry):
            for sp in range(2):
                s = 2 * s2 + sp
                for j in range(G):
                    b = j % 2
                    if j == 0:
                        issue_idx(1 - sp, rowbase + (s + 1) * G)
                    if j == 2:
                        _drain(idummy, srcb.at[1 - sp], sis[1 - sp])
                        _drain(idummy, dstb.at[1 - sp], sid_sem[1 - sp])
                    nb = 1 - b
                    nset, nrow = (sp, j + 1) if j < G - 1 else (1 - sp, 0)
                    issue_gather(nb, nset, nrow)
                    _drain(hdummy, xlv.at[b], sxl[b])
                    _drain(hdummy, xrv.at[b], sxr[b])
                    _drain(hdummy, msg.at[b], ssc[b])
                    @plsc.parallel_loop(0, K, unroll=1)
                    def _edges(e, _b=b):
                        edge_body(_b, e)
                    pltpu.async_copy(msg.at[b], acc.at[dstb.at[sp, j]],
                                     ssc[b], add=True)
            return carry

        lax.fori_loop(0, pairs, pair, 0)

        _drain(hdummy, xlv.at[0], sxl[0])
        _drain(hdummy, xrv.at[0], sxr[0])
        _drain(hdummy, msg.at[0], ssc[0])
        _drain(hdummy, msg.at[1], ssc[1])

        plsc.subcore_barrier()
        for j in range(ROWS_PER_TILE // K):
            sl = pl.ds(row0 + j * K, K)
            pltpu.sync_copy(acc.at[sl], out_hbm.at[cid].at[sl])

    return k


def _transform(x, Wl, Wr):
    Din, D = Wl.shape
    RB = 256

    def body(x_ref, wl_ref, wr_ref, xl_ref, xr_ref):
        xb = x_ref[...]
        xl_ref[...] = jnp.dot(xb, wl_ref[...], preferred_element_type=jnp.float32)
        xr_ref[...] = jnp.dot(xb, wr_ref[...], preferred_element_type=jnp.float32)

    return pl.pallas_call(
        body,
        grid=(NPAD // RB,),
        in_specs=[
            pl.BlockSpec((RB, Din), lambda i: (i, 0)),
            pl.BlockSpec((Din, D), lambda i: (0, 0)),
            pl.BlockSpec((Din, D), lambda i: (0, 0)),
        ],
        out_specs=[
            pl.BlockSpec((RB, D), lambda i: (i, 0)),
            pl.BlockSpec((RB, D), lambda i: (i, 0)),
        ],
        out_shape=[
            jax.ShapeDtypeStruct((NPAD, D), jnp.float32),
            jax.ShapeDtypeStruct((NPAD, D), jnp.float32),
        ],
    )(x, Wl, Wr)


def _combine_transform(acc0, acc1, expand, b, Wl, Wr):
    Dn = Wl.shape[1]
    RB = 256

    def body(a0_ref, a1_ref, ex_ref, b_ref, wl_ref, wr_ref, xl_ref, xr_ref):
        a0 = a0_ref[...]
        a1 = a1_ref[...]
        num = jnp.concatenate([a0[:, :64], a1[:, :64]], axis=1)
        den = jnp.concatenate([a0[:, 64:68], a1[:, 64:68]], axis=1)
        dexp = jnp.dot(den, ex_ref[...], preferred_element_type=jnp.float32)
        hh = num / (dexp + 1e-16) + b_ref[...]
        hh = jnp.where(hh > 0, hh, jnp.exp(jnp.minimum(hh, 0.0)) - 1.0)
        xl_ref[...] = jnp.dot(hh, wl_ref[...], preferred_element_type=jnp.float32)
        xr_ref[...] = jnp.dot(hh, wr_ref[...], preferred_element_type=jnp.float32)

    return pl.pallas_call(
        body,
        grid=(NPAD // RB,),
        in_specs=[
            pl.BlockSpec((RB, 128), lambda i: (i, 0)),
            pl.BlockSpec((RB, 128), lambda i: (i, 0)),
            pl.BlockSpec((H, D1), lambda i: (0, 0)),
            pl.BlockSpec((1, D1), lambda i: (0, 0)),
            pl.BlockSpec((D1, Dn), lambda i: (0, 0)),
            pl.BlockSpec((D1, Dn), lambda i: (0, 0)),
        ],
        out_specs=[
            pl.BlockSpec((RB, Dn), lambda i: (i, 0)),
            pl.BlockSpec((RB, Dn), lambda i: (i, 0)),
        ],
        out_shape=[
            jax.ShapeDtypeStruct((NPAD, Dn), jnp.float32),
            jax.ShapeDtypeStruct((NPAD, Dn), jnp.float32),
        ],
    )(acc0, acc1, expand, b.reshape(1, D1), Wl, Wr)


def _combine_final(acc0, acc1, b):
    RB = 256

    def body(a0_ref, a1_ref, b_ref, o_ref):
        a0 = a0_ref[...]
        a1 = a1_ref[...]
        num = a0[:, :OUT] + a1[:, :OUT]
        den = a0[:, OUT:OUT + 1] + a1[:, OUT:OUT + 1]
        dexp = jnp.broadcast_to(den, (RB, OUT))
        o_ref[...] = num / (dexp + 1e-16) + b_ref[...]

    return pl.pallas_call(
        body,
        grid=(NPAD // RB,),
        in_specs=[
            pl.BlockSpec((RB, 128), lambda i: (i, 0)),
            pl.BlockSpec((RB, 128), lambda i: (i, 0)),
            pl.BlockSpec((1, OUT), lambda i: (0, 0)),
        ],
        out_specs=pl.BlockSpec((RB, OUT), lambda i: (i, 0)),
        out_shape=jax.ShapeDtypeStruct((NPAD, OUT), jnp.float32),
    )(acc0, acc1, b.reshape(1, OUT))


def kernel(x, edge_index, Wl1, Wr1, att1, b1, Wl2, Wr2, att2, b2,
           Wl3, Wr3, att3, b3):
    loop = jnp.arange(N, dtype=edge_index.dtype)
    padv = N + jnp.arange(IDXROWS * K - EDGES, dtype=edge_index.dtype) % (NPAD - N)
    src = jnp.concatenate([edge_index[0], loop, padv]).reshape(IDXROWS, K)
    dst = jnp.concatenate([edge_index[1], loop, padv]).reshape(IDXROWS, K)
    x_pad = jnp.concatenate([x, jnp.zeros((NPAD - N, IN), x.dtype)], axis=0)

    expand8 = jnp.kron(jnp.eye(H, dtype=jnp.float32),
                       jnp.ones((1, HID), dtype=jnp.float32))

    edge_h = _make_edge_kernel(True, 8)
    edge_s = _make_edge_kernel(False, 8)

    xl, xr = _transform(x_pad, Wl1, Wr1)
    acc = edge_h(xl, xr, src, dst, att1.reshape(-1))
    xl, xr = _combine_transform(acc[0], acc[1], expand8, b1, Wl2, Wr2)
    acc = edge_h(xl, xr, src, dst, att2.reshape(-1))
    wpad = jnp.zeros((D1, D1 - OUT), jnp.float32)
    xl, xr = _combine_transform(acc[0], acc[1], expand8, b2,
                                jnp.concatenate([Wl3, wpad], axis=1),
                                jnp.concatenate([Wr3, wpad], axis=1))
    att3p = jnp.concatenate([att3.reshape(-1), jnp.zeros((64,), jnp.float32)])
    acc = edge_s(xl, xr, src, dst, att3p)
    out = _combine_final(acc[0], acc[1], b3)
    return out[:N]
